# baseline (speedup 1.0000x reference)
import jax
import jax.numpy as jnp
from jax.experimental import pallas as pl


_ROWS_PER_BLOCK = 2048


def _add_kernel(x_ref, pos_ref, o_ref):
    o_ref[...] = x_ref[...] + pos_ref[...]


def kernel(x, pos_table):
    B, T, D = x.shape
    x2 = x.reshape(B * T, D)
    rb = _ROWS_PER_BLOCK
    n_blocks = (B * T) // rb
    blocks_per_batch = T // rb

    out = pl.pallas_call(
        _add_kernel,
        grid=(n_blocks,),
        in_specs=[
            pl.BlockSpec((rb, D), lambda i: (i, 0)),
            pl.BlockSpec((rb, D), lambda i: (jax.lax.rem(i, blocks_per_batch), 0)),
        ],
        out_specs=pl.BlockSpec((rb, D), lambda i: (i, 0)),
        out_shape=jax.ShapeDtypeStruct((B * T, D), x.dtype),
    )(x2, pos_table)
    return out.reshape(B, T, D)


# speedup vs baseline: 1.2353x; 1.2353x over previous
import jax
import jax.numpy as jnp
from jax.experimental import pallas as pl


_ROWS_PER_BLOCK = 1024


def _add_kernel(x_ref, pos_ref, o_ref):
    o_ref[...] = x_ref[...] + pos_ref[...]


def kernel(x, pos_table):
    B, T, D = x.shape
    x2 = x.reshape(B * T, D)
    rb = _ROWS_PER_BLOCK
    blocks_per_batch = T // rb

    out = pl.pallas_call(
        _add_kernel,
        grid=(blocks_per_batch, B),
        in_specs=[
            pl.BlockSpec((rb, D), lambda i, b: (b * blocks_per_batch + i, 0)),
            pl.BlockSpec((rb, D), lambda i, b: (i, 0)),
        ],
        out_specs=pl.BlockSpec((rb, D), lambda i, b: (b * blocks_per_batch + i, 0)),
        out_shape=jax.ShapeDtypeStruct((B * T, D), x.dtype),
    )(x2, pos_table)
    return out.reshape(B, T, D)


# speedup vs baseline: 1.3139x; 1.0636x over previous
import jax
import jax.numpy as jnp
from jax.experimental import pallas as pl


_ROWS_PER_BLOCK = 2048


def _add_kernel(x_ref, pos_ref, o_ref):
    o_ref[...] = x_ref[...] + pos_ref[...]


def kernel(x, pos_table):
    B, T, D = x.shape
    x2 = x.reshape(B * T, D)
    rb = _ROWS_PER_BLOCK
    blocks_per_batch = T // rb

    out = pl.pallas_call(
        _add_kernel,
        grid=(blocks_per_batch, B),
        in_specs=[
            pl.BlockSpec((rb, D), lambda i, b: (b * blocks_per_batch + i, 0)),
            pl.BlockSpec((rb, D), lambda i, b: (i, 0)),
        ],
        out_specs=pl.BlockSpec((rb, D), lambda i, b: (b * blocks_per_batch + i, 0)),
        out_shape=jax.ShapeDtypeStruct((B * T, D), x.dtype),
    )(x2, pos_table)
    return out.reshape(B, T, D)
